# Initial kernel scaffold; baseline (speedup 1.0000x reference)
import jax
import jax.numpy as jnp
from jax.experimental import pallas as pl


def kernel(x, l1_Wq, l1_bq, l1_Wk, l1_bk, l1_Wv, l1_bv, l1_convW, l1_convb, l2_Wq, l2_bq, l2_Wk, l2_bk, l2_Wv, l2_bv, l2_convW, l2_convb, fc1_W, fc1_b, fc2_W, fc2_b):
    raise NotImplementedError("write your pallas kernel here")


# speedup vs baseline: 20.7833x; 20.7833x over previous
import functools

import jax
import jax.numpy as jnp
from jax.experimental import pallas as pl
from jax.experimental.pallas import tpu as pltpu

K = 9
SCALE = 2
N = 256
NEG = -1e30
HI = jax.lax.Precision.HIGHEST
BF = jnp.bfloat16


def _attn_layer_kernel(xf_ref, wq_ref, bq_ref, wk_ref, bk_ref, wv_ref, bv_ref,
                       convw_ref, convb_ref, out_ref, *, BB, C, COUT,
                       c_outer):
    Xb = xf_ref[...].reshape(BB * C, N).astype(BF)
    dn = (((1,), (1,)), ((), ()))
    Q = jax.lax.dot_general(Xb, wq_ref[...], dn,
                            preferred_element_type=jnp.float32) + bq_ref[...]
    Km = jax.lax.dot_general(Xb, wk_ref[...], dn,
                             preferred_element_type=jnp.float32) + bk_ref[...]
    V = jax.lax.dot_general(Xb, wv_ref[...], dn,
                            preferred_element_type=jnp.float32) + bv_ref[...]
    Qb = Q.astype(BF)
    Kb = Km.astype(BF)

    iota_sub = jax.lax.broadcasted_iota(jnp.int32, (N, N), 0)

    for i in range(BB):
        v_i = V[i * C:(i + 1) * C, :]
        simT = jax.lax.dot_general(Kb[i * C:(i + 1) * C, :],
                                   Qb[i * C:(i + 1) * C, :],
                                   (((0,), (0,)), ((), ())),
                                   preferred_element_type=jnp.float32)
        running = simT
        mx0 = None
        es = []
        parts = []
        for kk in range(K):
            mx = jnp.max(running, axis=0, keepdims=True)
            if kk == 0:
                mx0 = mx
                e = jnp.ones((1, N), jnp.float32)
            else:
                e = jnp.exp(mx - mx0)
            es.append(e)
            hit = running == mx
            idxk = jnp.min(jnp.where(hit, iota_sub, N), axis=0,
                           keepdims=True)
            sel = iota_sub == idxk
            PT = jnp.where(sel, 1.0, 0.0)
            parts.append(
                jax.lax.dot_general(v_i, PT, (((1,), (0,)), ((), ())),
                                    preferred_element_type=jnp.float32,
                                    precision=HI))
            running = jnp.where(sel, NEG, running)
        denom = (((es[8] + es[7]) + (es[6] + es[5]))
                 + ((es[4] + es[3]) + (es[2] + es[1]))) + es[0]
        recip = pl.reciprocal(denom, approx=True)
        nbw = [(p * (e * recip)).astype(BF) for p, e in zip(parts, es)]

        if c_outer:
            nb3 = jnp.concatenate([p.reshape(1, C, N) for p in nbw], axis=0)
            out_i = None
            for c in range(C):
                d = jax.lax.dot_general(convw_ref[c], nb3[:, c, :],
                                        (((1,), (0,)), ((), ())),
                                        preferred_element_type=jnp.float32)
                out_i = d if out_i is None else out_i + d
        else:
            out_i = None
            for kk in range(K):
                d = jax.lax.dot_general(convw_ref[kk], nbw[kk],
                                        (((1,), (0,)), ((), ())),
                                        preferred_element_type=jnp.float32)
                out_i = d if out_i is None else out_i + d
        out_ref[i] = jnp.maximum(out_i + convb_ref[...], 0.0)


def _attn_layer(xf, Wq, bq, Wk, bk, Wv, bv, convW, convb, BB, c_outer):
    B, C, _ = xf.shape
    COUT = convW.shape[0]
    if c_outer:
        convWp = convW.transpose(1, 0, 2).astype(BF)
        wspec = pl.BlockSpec((C, COUT, K), lambda i: (0, 0, 0))
    else:
        convWp = convW.transpose(2, 0, 1).astype(BF)
        wspec = pl.BlockSpec((K, COUT, C), lambda i: (0, 0, 0))
    kern = functools.partial(_attn_layer_kernel, BB=BB, C=C, COUT=COUT,
                             c_outer=c_outer)
    return pl.pallas_call(
        kern,
        grid=(B // BB,),
        in_specs=[
            pl.BlockSpec((BB, C, N), lambda i: (i, 0, 0)),
            pl.BlockSpec((N, N), lambda i: (0, 0)),
            pl.BlockSpec((1, N), lambda i: (0, 0)),
            pl.BlockSpec((N, N), lambda i: (0, 0)),
            pl.BlockSpec((1, N), lambda i: (0, 0)),
            pl.BlockSpec((N, N), lambda i: (0, 0)),
            pl.BlockSpec((1, N), lambda i: (0, 0)),
            wspec,
            pl.BlockSpec((COUT, 1), lambda i: (0, 0)),
        ],
        out_specs=pl.BlockSpec((BB, COUT, N), lambda i: (i, 0, 0)),
        out_shape=jax.ShapeDtypeStruct((B, COUT, N), jnp.float32),
    )(xf, Wq.astype(BF), bq.reshape(1, N), Wk.astype(BF), bk.reshape(1, N),
      Wv.astype(BF), bv.reshape(1, N), convWp, convb.reshape(COUT, 1))


def _fc_kernel(p_ref, w1_ref, b1_ref, w2_ref, b2_ref, out_ref, acc_ref, *, NK):
    k = pl.program_id(0)

    @pl.when(k == 0)
    def _():
        acc_ref[...] = jnp.zeros_like(acc_ref)

    acc_ref[...] += jax.lax.dot_general(
        p_ref[...], w1_ref[...], (((1,), (1,)), ((), ())),
        preferred_element_type=jnp.float32)

    @pl.when(k == NK - 1)
    def _():
        h = jnp.maximum(acc_ref[...] + b1_ref[...], 0.0).astype(BF)
        out_ref[...] = jax.lax.dot_general(
            h, w2_ref[...], (((1,), (1,)), ((), ())),
            preferred_element_type=jnp.float32) + b2_ref[...]


def _fc_head(p, fc1_W, fc1_b, fc2_W, fc2_b):
    B, D = p.shape
    H = fc1_W.shape[0]
    O = fc2_W.shape[0]
    NK = 16
    BK = D // NK
    kern = functools.partial(_fc_kernel, NK=NK)
    return pl.pallas_call(
        kern,
        grid=(NK,),
        in_specs=[
            pl.BlockSpec((B, BK), lambda k: (0, k)),
            pl.BlockSpec((H, BK), lambda k: (0, k)),
            pl.BlockSpec((1, H), lambda k: (0, 0)),
            pl.BlockSpec((O, H), lambda k: (0, 0)),
            pl.BlockSpec((1, O), lambda k: (0, 0)),
        ],
        out_specs=pl.BlockSpec((B, O), lambda k: (0, 0)),
        out_shape=jax.ShapeDtypeStruct((B, O), jnp.float32),
        scratch_shapes=[pltpu.VMEM((B, H), jnp.float32)],
    )(p.astype(BF), fc1_W.astype(BF), fc1_b.reshape(1, H),
      fc2_W.astype(BF), fc2_b.reshape(1, O))


def kernel(x, l1_Wq, l1_bq, l1_Wk, l1_bk, l1_Wv, l1_bv, l1_convW, l1_convb,
           l2_Wq, l2_bq, l2_Wk, l2_bk, l2_Wv, l2_bv, l2_convW, l2_convb,
           fc1_W, fc1_b, fc2_W, fc2_b):
    B = x.shape[0]
    r = SCALE
    Bx, Cx, H, W = x.shape
    xs = x.reshape(Bx, Cx, H // r, r, W // r, r).transpose(0, 1, 3, 5, 2, 4)
    xf1 = xs.reshape(Bx, Cx * r * r, (H // r) * (W // r))

    a1 = _attn_layer(xf1, l1_Wq, l1_bq, l1_Wk, l1_bk, l1_Wv, l1_bv,
                     l1_convW, l1_convb, BB=8, c_outer=True)
    a2 = _attn_layer(a1, l2_Wq, l2_bq, l2_Wk, l2_bk, l2_Wv, l2_bv,
                     l2_convW, l2_convb, BB=8, c_outer=False)

    h = a2.reshape(B, 32, 2, 2, 16, 16).transpose(0, 1, 4, 2, 5, 3)
    p = h.reshape(B, 32 * 32 * 32)

    return _fc_head(p, fc1_W, fc1_b, fc2_W, fc2_b)
